# baseline (speedup 1.0000x reference)
import jax
import jax.numpy as jnp
from jax.experimental import pallas as pl
from jax.experimental.pallas import tpu as pltpu

TOP_K = 8
NUM_EXPERTS = 64
HIDDEN_DIM = 4096

NBUF = 5
NPRIME = 4
CHUNK = 512
SLICE = 128


def _router_body(x_hbm, wt_ref, probs_ref, scores_ref, idx_ref, buf, logit_scr, sem):
    n_tokens = x_hbm.shape[0]
    n_chunks = n_tokens // CHUNK

    def start(i):
        pltpu.make_async_copy(
            x_hbm.at[pl.ds(i * CHUNK, CHUNK), :], buf.at[i % NBUF], sem.at[i % NBUF]
        ).start()

    def wait(i):
        pltpu.make_async_copy(
            x_hbm.at[pl.ds(i * CHUNK, CHUNK), :], buf.at[i % NBUF], sem.at[i % NBUF]
        ).wait()

    iota = jax.lax.broadcasted_iota(jnp.int32, (SLICE, NUM_EXPERTS), 1).astype(
        jnp.float32
    )

    for i in range(NPRIME):
        start(i)

    for i in range(n_chunks):
        wait(i)
        if i + NPRIME < n_chunks:
            start(i + NPRIME)
        x = buf[i % NBUF]
        logit_scr[...] = jax.lax.dot_general(
            x, wt_ref[...], (((1,), (0,)), ((), ())),
            preferred_element_type=jnp.float32,
        )
        for t in range(CHUNK // SLICE):
            logits = logit_scr[pl.ds(t * SLICE, SLICE), :]
            m = jnp.max(logits, axis=-1, keepdims=True)
            e = jnp.exp(logits - m)
            s = jnp.sum(e, axis=-1, keepdims=True)
            probs = e / s
            rows = pl.ds(i * CHUNK + t * SLICE, SLICE)
            probs_ref[rows, :] = probs

            work = probs
            vals = []
            idxs = []
            for _ in range(TOP_K):
                mj = jnp.max(work, axis=-1, keepdims=True)
                amj = jnp.min(
                    jnp.where(work == mj, iota, float(NUM_EXPERTS)),
                    axis=-1,
                    keepdims=True,
                )
                vals.append(mj)
                idxs.append(amj)
                work = jnp.where(iota == amj, -1.0, work)
            v = jnp.concatenate(vals, axis=1)
            scores_ref[rows, :] = v / jnp.sum(v, axis=1, keepdims=True)
            idx_ref[rows, :] = jnp.concatenate(idxs, axis=1).astype(jnp.int32)


def kernel(hidden_states, weight):
    x = hidden_states.reshape(-1, HIDDEN_DIM)
    wt = weight.T
    n_tokens = x.shape[0]
    probs, scores, idx = pl.pallas_call(
        _router_body,
        in_specs=[
            pl.BlockSpec(memory_space=pl.ANY),
            pl.BlockSpec(memory_space=pltpu.MemorySpace.VMEM),
        ],
        out_specs=[
            pl.BlockSpec(memory_space=pltpu.MemorySpace.VMEM),
            pl.BlockSpec(memory_space=pltpu.MemorySpace.VMEM),
            pl.BlockSpec(memory_space=pltpu.MemorySpace.VMEM),
        ],
        out_shape=[
            jax.ShapeDtypeStruct((n_tokens, NUM_EXPERTS), jnp.float32),
            jax.ShapeDtypeStruct((n_tokens, TOP_K), jnp.float32),
            jax.ShapeDtypeStruct((n_tokens, TOP_K), jnp.int32),
        ],
        scratch_shapes=[
            pltpu.VMEM((NBUF, CHUNK, HIDDEN_DIM), jnp.float32),
            pltpu.VMEM((CHUNK, NUM_EXPERTS), jnp.float32),
            pltpu.SemaphoreType.DMA((NBUF,)),
        ],
    )(x, wt)
    return probs, scores, idx


# speedup vs baseline: 1.0021x; 1.0021x over previous
import jax
import jax.numpy as jnp
from jax.experimental import pallas as pl
from jax.experimental.pallas import tpu as pltpu

TOP_K = 8
NUM_EXPERTS = 64
HIDDEN_DIM = 4096

NBUF = 5
NPRIME = 4
CHUNK = 512
SLICE = 64


def _router_body(x_hbm, wt_ref, probs_ref, scores_ref, idx_ref, buf, logit_scr, sem):
    n_tokens = x_hbm.shape[0]
    n_chunks = n_tokens // CHUNK

    def start(i):
        pltpu.make_async_copy(
            x_hbm.at[pl.ds(i * CHUNK, CHUNK), :], buf.at[i % NBUF], sem.at[i % NBUF]
        ).start()

    def wait(i):
        pltpu.make_async_copy(
            x_hbm.at[pl.ds(i * CHUNK, CHUNK), :], buf.at[i % NBUF], sem.at[i % NBUF]
        ).wait()

    iota = jax.lax.broadcasted_iota(jnp.int32, (SLICE, NUM_EXPERTS), 1).astype(
        jnp.float32
    )

    for i in range(NPRIME):
        start(i)

    for i in range(n_chunks):
        wait(i)
        if i + NPRIME < n_chunks:
            start(i + NPRIME)
        x = buf[i % NBUF]
        logit_scr[...] = jax.lax.dot_general(
            x, wt_ref[...], (((1,), (0,)), ((), ())),
            preferred_element_type=jnp.float32,
        )
        for t in range(CHUNK // SLICE):
            logits = logit_scr[pl.ds(t * SLICE, SLICE), :]
            m = jnp.max(logits, axis=-1, keepdims=True)
            e = jnp.exp(logits - m)
            s = jnp.sum(e, axis=-1, keepdims=True)
            probs = e / s
            rows = pl.ds(i * CHUNK + t * SLICE, SLICE)
            probs_ref[rows, :] = probs

            work = probs
            vals = []
            idxs = []
            for _ in range(TOP_K):
                mj = jnp.max(work, axis=-1, keepdims=True)
                amj = jnp.min(
                    jnp.where(work == mj, iota, float(NUM_EXPERTS)),
                    axis=-1,
                    keepdims=True,
                )
                vals.append(mj)
                idxs.append(amj)
                work = jnp.where(iota == amj, -1.0, work)
            v = jnp.concatenate(vals, axis=1)
            scores_ref[rows, :] = v / jnp.sum(v, axis=1, keepdims=True)
            idx_ref[rows, :] = jnp.concatenate(idxs, axis=1).astype(jnp.int32)


def kernel(hidden_states, weight):
    x = hidden_states.reshape(-1, HIDDEN_DIM)
    wt = weight.T
    n_tokens = x.shape[0]
    probs, scores, idx = pl.pallas_call(
        _router_body,
        in_specs=[
            pl.BlockSpec(memory_space=pl.ANY),
            pl.BlockSpec(memory_space=pltpu.MemorySpace.VMEM),
        ],
        out_specs=[
            pl.BlockSpec(memory_space=pltpu.MemorySpace.VMEM),
            pl.BlockSpec(memory_space=pltpu.MemorySpace.VMEM),
            pl.BlockSpec(memory_space=pltpu.MemorySpace.VMEM),
        ],
        out_shape=[
            jax.ShapeDtypeStruct((n_tokens, NUM_EXPERTS), jnp.float32),
            jax.ShapeDtypeStruct((n_tokens, TOP_K), jnp.float32),
            jax.ShapeDtypeStruct((n_tokens, TOP_K), jnp.int32),
        ],
        scratch_shapes=[
            pltpu.VMEM((NBUF, CHUNK, HIDDEN_DIM), jnp.float32),
            pltpu.VMEM((CHUNK, NUM_EXPERTS), jnp.float32),
            pltpu.SemaphoreType.DMA((NBUF,)),
        ],
    )(x, wt)
    return probs, scores, idx


# speedup vs baseline: 1.3933x; 1.3904x over previous
import jax
import jax.numpy as jnp
from jax.experimental import pallas as pl
from jax.experimental.pallas import tpu as pltpu

TOP_K = 8
NUM_EXPERTS = 64
HIDDEN_DIM = 4096

NBUF = 5
NPRIME = 4
CHUNK = 512


def _router_body(x_hbm, w_ref, probs_ref, scores_ref, idx_ref, buf, sem):
    n_tokens = x_hbm.shape[0]
    n_chunks = n_tokens // CHUNK

    def start(i):
        pltpu.make_async_copy(
            x_hbm.at[pl.ds(i * CHUNK, CHUNK), :], buf.at[i % NBUF], sem.at[i % NBUF]
        ).start()

    def wait(i):
        pltpu.make_async_copy(
            x_hbm.at[pl.ds(i * CHUNK, CHUNK), :], buf.at[i % NBUF], sem.at[i % NBUF]
        ).wait()

    iota_t = jax.lax.broadcasted_iota(jnp.int32, (NUM_EXPERTS, CHUNK), 0).astype(
        jnp.float32
    )

    for i in range(NPRIME):
        start(i)

    for i in range(n_chunks):
        wait(i)
        if i + NPRIME < n_chunks:
            start(i + NPRIME)
        x = buf[i % NBUF]
        logits_t = jax.lax.dot_general(
            w_ref[...], x, (((1,), (1,)), ((), ())),
            preferred_element_type=jnp.float32,
        )
        m = jnp.max(logits_t, axis=0, keepdims=True)
        e = jnp.exp(logits_t - m)
        s = jnp.sum(e, axis=0, keepdims=True)
        probs_t = e / s
        rows = pl.ds(i * CHUNK, CHUNK)
        probs_ref[rows, :] = probs_t.T

        work = probs_t
        vals = []
        idxs = []
        for _ in range(TOP_K):
            mj = jnp.max(work, axis=0, keepdims=True)
            amj = jnp.min(
                jnp.where(work == mj, iota_t, float(NUM_EXPERTS)),
                axis=0,
                keepdims=True,
            )
            vals.append(mj)
            idxs.append(amj)
            work = jnp.where(iota_t == amj, -1.0, work)
        v_t = jnp.concatenate(vals, axis=0)
        scores_t = v_t / jnp.sum(v_t, axis=0, keepdims=True)
        scores_ref[rows, :] = scores_t.T
        idx_ref[rows, :] = jnp.concatenate(idxs, axis=0).T.astype(jnp.int32)


def kernel(hidden_states, weight):
    x = hidden_states.reshape(-1, HIDDEN_DIM)
    n_tokens = x.shape[0]
    probs, scores, idx = pl.pallas_call(
        _router_body,
        in_specs=[
            pl.BlockSpec(memory_space=pl.ANY),
            pl.BlockSpec(memory_space=pltpu.MemorySpace.VMEM),
        ],
        out_specs=[
            pl.BlockSpec(memory_space=pltpu.MemorySpace.VMEM),
            pl.BlockSpec(memory_space=pltpu.MemorySpace.VMEM),
            pl.BlockSpec(memory_space=pltpu.MemorySpace.VMEM),
        ],
        out_shape=[
            jax.ShapeDtypeStruct((n_tokens, NUM_EXPERTS), jnp.float32),
            jax.ShapeDtypeStruct((n_tokens, TOP_K), jnp.float32),
            jax.ShapeDtypeStruct((n_tokens, TOP_K), jnp.int32),
        ],
        scratch_shapes=[
            pltpu.VMEM((NBUF, CHUNK, HIDDEN_DIM), jnp.float32),
            pltpu.SemaphoreType.DMA((NBUF,)),
        ],
    )(x, weight)
    return probs, scores, idx


# speedup vs baseline: 1.5771x; 1.1319x over previous
import jax
import jax.numpy as jnp
from jax.experimental import pallas as pl
from jax.experimental.pallas import tpu as pltpu

TOP_K = 8
NUM_EXPERTS = 64
HIDDEN_DIM = 4096

NBUF = 3
NPRIME = 2
CHUNK = 1024


def _router_body(x_hbm, w_ref, probs_ref, scores_ref, idx_ref, buf, sem):
    n_tokens = x_hbm.shape[0]
    n_chunks = n_tokens // CHUNK

    def start(i):
        pltpu.make_async_copy(
            x_hbm.at[pl.ds(i * CHUNK, CHUNK), :], buf.at[i % NBUF], sem.at[i % NBUF]
        ).start()

    def wait(i):
        pltpu.make_async_copy(
            x_hbm.at[pl.ds(i * CHUNK, CHUNK), :], buf.at[i % NBUF], sem.at[i % NBUF]
        ).wait()

    iota_t = jax.lax.broadcasted_iota(jnp.int32, (NUM_EXPERTS, CHUNK), 0).astype(
        jnp.float32
    )

    for i in range(NPRIME):
        start(i)

    for i in range(n_chunks):
        wait(i)
        if i + NPRIME < n_chunks:
            start(i + NPRIME)
        x = buf[i % NBUF]
        logits_t = jax.lax.dot_general(
            w_ref[...], x, (((1,), (1,)), ((), ())),
            preferred_element_type=jnp.float32,
        )
        m = jnp.max(logits_t, axis=0, keepdims=True)
        e = jnp.exp(logits_t - m)
        s = jnp.sum(e, axis=0, keepdims=True)
        probs_t = e / s
        rows = pl.ds(i * CHUNK, CHUNK)
        probs_ref[rows, :] = probs_t.T

        work = probs_t
        vals = []
        idxs = []
        for _ in range(TOP_K):
            mj = jnp.max(work, axis=0, keepdims=True)
            amj = jnp.min(
                jnp.where(work == mj, iota_t, float(NUM_EXPERTS)),
                axis=0,
                keepdims=True,
            )
            vals.append(mj)
            idxs.append(amj)
            work = jnp.where(iota_t == amj, -1.0, work)
        v_t = jnp.concatenate(vals, axis=0)
        scores_ref[:, rows] = v_t / jnp.sum(v_t, axis=0, keepdims=True)
        idx_ref[:, rows] = jnp.concatenate(idxs, axis=0).astype(jnp.int32)


def kernel(hidden_states, weight):
    x = hidden_states.reshape(-1, HIDDEN_DIM)
    n_tokens = x.shape[0]
    probs, scores_t, idx_t = pl.pallas_call(
        _router_body,
        in_specs=[
            pl.BlockSpec(memory_space=pl.ANY),
            pl.BlockSpec(memory_space=pltpu.MemorySpace.VMEM),
        ],
        out_specs=[
            pl.BlockSpec(memory_space=pltpu.MemorySpace.VMEM),
            pl.BlockSpec(memory_space=pltpu.MemorySpace.VMEM),
            pl.BlockSpec(memory_space=pltpu.MemorySpace.VMEM),
        ],
        out_shape=[
            jax.ShapeDtypeStruct((n_tokens, NUM_EXPERTS), jnp.float32),
            jax.ShapeDtypeStruct((TOP_K, n_tokens), jnp.float32),
            jax.ShapeDtypeStruct((TOP_K, n_tokens), jnp.int32),
        ],
        scratch_shapes=[
            pltpu.VMEM((NBUF, CHUNK, HIDDEN_DIM), jnp.float32),
            pltpu.SemaphoreType.DMA((NBUF,)),
        ],
    )(x, weight)
    return probs, scores_t.T, idx_t.T
